# Initial kernel scaffold; baseline (speedup 1.0000x reference)
import jax
import jax.numpy as jnp
from jax.experimental import pallas as pl


def kernel(vals, distances):
    raise NotImplementedError("write your pallas kernel here")


# speedup vs baseline: 117.3028x; 117.3028x over previous
import functools

import jax
import jax.numpy as jnp
from jax import lax
from jax.experimental import pallas as pl
from jax.experimental.pallas import tpu as pltpu
from jax.experimental.pallas import tpu_sc as plsc

K = 32
B = 4
S = 4096
T = B * S
LANES = 16

NUM_CORES = 2
NUM_SUBCORES = 16
NW = NUM_CORES * NUM_SUBCORES
TOK_PER_W = T // NW
CHUNK = 128
N_CHUNKS = TOK_PER_W // CHUNK
GROUPS = CHUNK // LANES


def _sc_body(vals_hbm, dist_hbm, out_hbm, vals_v, dist_v, stage_v):
    wid = lax.axis_index("s") * NUM_CORES + lax.axis_index("c")
    base = wid * TOK_PER_W
    lane_iota = lax.iota(jnp.int32, LANES)

    def chunk_body(ci, carry):
        tok0 = base + ci * CHUNK
        pltpu.sync_copy(vals_hbm.at[pl.ds(tok0, CHUNK)], vals_v)
        pltpu.sync_copy(dist_hbm.at[pl.ds(tok0, CHUNK)], dist_v)

        def group_body(g, carry):
            r0 = g * LANES
            rows = r0 + lane_iota
            for t in range(LANES):
                for h in range(2):
                    stage_v[r0 + t, pl.ds(h * LANES, LANES)] = (
                        dist_v[r0 + t, pl.ds(h * LANES, LANES)])
            count = jnp.zeros((LANES,), jnp.int32)
            prev = []
            for j in range(K):
                vj = plsc.load_gather(
                    vals_v, [rows, jnp.full((LANES,), j, jnp.int32)])
                dup = jnp.zeros((LANES,), jnp.bool_)
                for vl in prev:
                    dup = dup | (vl == vj)
                new = (vj != 0) & jnp.logical_not(dup)
                count = count + new.astype(jnp.int32)
                plsc.store_scatter(
                    stage_v,
                    [rows, jnp.full((LANES,), K + j, jnp.int32)],
                    count.astype(jnp.float32))
                prev.append(vj)
            return carry

        lax.fori_loop(0, GROUPS, group_body, 0)
        pltpu.sync_copy(stage_v, out_hbm.at[pl.ds(tok0, CHUNK)])
        return carry

    lax.fori_loop(0, N_CHUNKS, chunk_body, 0)


@functools.partial(jax.jit, static_argnames=())
def kernel(vals, distances):
    vals2 = vals.reshape(T, K)
    dist2 = distances.reshape(T, K)
    mesh = plsc.VectorSubcoreMesh(
        core_axis_name="c", subcore_axis_name="s",
        num_cores=NUM_CORES, num_subcores=NUM_SUBCORES)
    out = pl.kernel(
        _sc_body,
        out_type=jax.ShapeDtypeStruct((T, 2 * K), jnp.float32),
        mesh=mesh,
        scratch_types=[
            pltpu.VMEM((CHUNK, K), jnp.int32),
            pltpu.VMEM((CHUNK, K), jnp.float32),
            pltpu.VMEM((CHUNK, 2 * K), jnp.float32),
        ],
        compiler_params=pltpu.CompilerParams(needs_layout_passes=False),
    )(vals2, dist2)
    return out.reshape(B, S, 2 * K)


# speedup vs baseline: 132.0757x; 1.1259x over previous
import functools

import jax
import jax.numpy as jnp
from jax import lax
from jax.experimental import pallas as pl
from jax.experimental.pallas import tpu as pltpu
from jax.experimental.pallas import tpu_sc as plsc

K = 32
B = 4
S = 4096
T = B * S
LANES = 16

NUM_CORES = 2
NUM_SUBCORES = 16
NW = NUM_CORES * NUM_SUBCORES
TOK_PER_W = T // NW
CHUNK = 128
N_CHUNKS = TOK_PER_W // CHUNK
GROUPS = CHUNK // LANES


def _sc_body(vals_hbm, dist_hbm, out_hbm, vals_v, dist_v, stage_v):
    wid = lax.axis_index("s") * NUM_CORES + lax.axis_index("c")
    base = wid * TOK_PER_W
    lane_iota = lax.iota(jnp.int32, LANES)

    def chunk_body(ci, carry):
        tok0 = base + ci * CHUNK
        pltpu.sync_copy(vals_hbm.at[pl.ds(tok0, CHUNK)], vals_v)
        pltpu.sync_copy(dist_hbm.at[pl.ds(tok0, CHUNK)], dist_v)

        def group_body(g, carry):
            r0 = g * LANES
            rows = r0 + lane_iota
            for t in range(LANES):
                for h in range(2):
                    stage_v[r0 + t, pl.ds(h * LANES, LANES)] = (
                        dist_v[r0 + t, pl.ds(h * LANES, LANES)])
            cols = [
                plsc.load_gather(
                    vals_v, [rows, jnp.full((LANES,), j, jnp.int32)])
                for j in range(K)
            ]
            count = jnp.zeros((LANES,), jnp.int32)
            for j in range(K):
                vj = cols[j]
                terms = [cols[l] == vj for l in range(j)]
                while len(terms) > 1:
                    nxt = []
                    for i in range(0, len(terms) - 1, 2):
                        nxt.append(terms[i] | terms[i + 1])
                    if len(terms) % 2:
                        nxt.append(terms[-1])
                    terms = nxt
                new = vj != 0
                if terms:
                    new = new & jnp.logical_not(terms[0])
                count = count + new.astype(jnp.int32)
                plsc.store_scatter(
                    stage_v,
                    [rows, jnp.full((LANES,), K + j, jnp.int32)],
                    count.astype(jnp.float32))
            return carry

        lax.fori_loop(0, GROUPS, group_body, 0)
        pltpu.sync_copy(stage_v, out_hbm.at[pl.ds(tok0, CHUNK)])
        return carry

    lax.fori_loop(0, N_CHUNKS, chunk_body, 0)


@functools.partial(jax.jit, static_argnames=())
def kernel(vals, distances):
    vals2 = vals.reshape(T, K)
    dist2 = distances.reshape(T, K)
    mesh = plsc.VectorSubcoreMesh(
        core_axis_name="c", subcore_axis_name="s",
        num_cores=NUM_CORES, num_subcores=NUM_SUBCORES)
    out = pl.kernel(
        _sc_body,
        out_type=jax.ShapeDtypeStruct((T, 2 * K), jnp.float32),
        mesh=mesh,
        scratch_types=[
            pltpu.VMEM((CHUNK, K), jnp.int32),
            pltpu.VMEM((CHUNK, K), jnp.float32),
            pltpu.VMEM((CHUNK, 2 * K), jnp.float32),
        ],
        compiler_params=pltpu.CompilerParams(needs_layout_passes=False),
    )(vals2, dist2)
    return out.reshape(B, S, 2 * K)
